# baseline (speedup 1.0000x reference)
import functools

import jax
import jax.numpy as jnp
from jax import lax
from jax.experimental import pallas as pl
from jax.experimental.pallas import tpu as pltpu
from jax.experimental.pallas import tpu_sc as plsc

_H = 128


def _silu(x):
    return x * jax.nn.sigmoid(x)


def _rows(b):
    return pl.BlockSpec((b, _H), lambda i: (i, 0))


def _bcast(shape):
    return pl.BlockSpec(shape, lambda i: (0,) * len(shape))


def _dot(a, b):
    return jnp.dot(a, b, preferred_element_type=jnp.float32)


def _enc_node_body(x_ref, w1, b1, w2, b2, h_ref, hb_ref):
    u = _silu(_dot(x_ref[:], w1[:]) + b1[:])
    h = _dot(u, w2[:]) + b2[:]
    h_ref[:] = h
    hb_ref[:] = h.astype(jnp.bfloat16)


def _encode_nodes(x, w1, b1, w2, b2, bn):
    n, nin = x.shape
    return pl.pallas_call(
        _enc_node_body,
        grid=(n // bn,),
        in_specs=[pl.BlockSpec((bn, nin), lambda i: (i, 0)),
                  _bcast((nin, _H)), _bcast((1, _H)),
                  _bcast((_H, _H)), _bcast((1, _H))],
        out_specs=[_rows(bn), _rows(bn)],
        out_shape=[jax.ShapeDtypeStruct((n, _H), jnp.float32),
                   jax.ShapeDtypeStruct((n, _H), jnp.bfloat16)],
    )(x, w1, b1, w2, b2)


def _enc_edge_body(a_ref, w1, b1, w2, b2, e_ref):
    u = _silu(_dot(a_ref[:], w1[:]) + b1[:])
    e_ref[:] = _dot(u, w2[:]) + b2[:]


def _encode_edges(a, w1, b1, w2, b2, be):
    e, ein = a.shape
    return pl.pallas_call(
        _enc_edge_body,
        grid=(e // be,),
        in_specs=[pl.BlockSpec((be, ein), lambda i: (i, 0)),
                  _bcast((ein, _H)), _bcast((1, _H)),
                  _bcast((_H, _H)), _bcast((1, _H))],
        out_specs=_rows(be),
        out_shape=jax.ShapeDtypeStruct((e, _H), jnp.float32),
    )(a, w1, b1, w2, b2)


def _edge_blk_body(e_ref, hs_ref, hd_ref, w1, b1, w2, b2, eup_ref, enew_ref):
    e = e_ref[:]
    e_in = jnp.concatenate([e, hs_ref[:].astype(jnp.float32),
                            hd_ref[:].astype(jnp.float32)], axis=-1)
    u = _silu(_dot(e_in, w1[:]) + b1[:])
    eup = _dot(u, w2[:]) + b2[:]
    eup_ref[:] = eup
    enew_ref[:] = e + eup


def _edge_block(e, hs, hd, w1, b1, w2, b2, be):
    ne = e.shape[0]
    return pl.pallas_call(
        _edge_blk_body,
        grid=(ne // be,),
        in_specs=[_rows(be), _rows(be), _rows(be),
                  _bcast((3 * _H, _H)), _bcast((1, _H)),
                  _bcast((_H, _H)), _bcast((1, _H))],
        out_specs=[_rows(be), _rows(be)],
        out_shape=[jax.ShapeDtypeStruct((ne, _H), jnp.float32)] * 2,
    )(e, hs, hd, w1, b1, w2, b2)


def _node_blk_body(h_ref, agg_ref, w1, b1, w2, b2, hn_ref, hb_ref):
    h = h_ref[:]
    x_in = jnp.concatenate([h, agg_ref[:]], axis=-1)
    u = _silu(_dot(x_in, w1[:]) + b1[:])
    hn = h + _dot(u, w2[:]) + b2[:]
    hn_ref[:] = hn
    hb_ref[:] = hn.astype(jnp.bfloat16)


def _node_block(h, agg, w1, b1, w2, b2, bn):
    n = h.shape[0]
    return pl.pallas_call(
        _node_blk_body,
        grid=(n // bn,),
        in_specs=[_rows(bn), _rows(bn),
                  _bcast((2 * _H, _H)), _bcast((1, _H)),
                  _bcast((_H, _H)), _bcast((1, _H))],
        out_specs=[_rows(bn), _rows(bn)],
        out_shape=[jax.ShapeDtypeStruct((n, _H), jnp.float32),
                   jax.ShapeDtypeStruct((n, _H), jnp.bfloat16)],
    )(h, agg, w1, b1, w2, b2)


def _dec_body(h_ref, w1, b1, w2, b2, o_ref):
    u = _silu(_dot(h_ref[:], w1[:]) + b1[:])
    o_ref[:] = _dot(u, w2[:]) + b2[:]


def _decode(h, w1, b1, w2p, b2p, bn):
    n = h.shape[0]
    return pl.pallas_call(
        _dec_body,
        grid=(n // bn,),
        in_specs=[_rows(bn),
                  _bcast((_H, _H)), _bcast((1, _H)),
                  _bcast((_H, _H)), _bcast((1, _H))],
        out_specs=_rows(bn),
        out_shape=jax.ShapeDtypeStruct((n, _H), jnp.float32),
    )(h, w1, b1, w2p, b2p)


_SCK = 80
_NC = 2
_NS = 16


def _sc_gather(h, src, dst):
    ne = src.shape[0]
    nw = _NC * _NS
    per_w = ne // nw
    nch = per_w // _SCK
    dt = h.dtype
    mesh = plsc.VectorSubcoreMesh(core_axis_name="c", subcore_axis_name="s")

    @functools.partial(
        pl.kernel, mesh=mesh,
        compiler_params=pltpu.CompilerParams(use_tc_tiling_on_sc=False),
        out_type=[jax.ShapeDtypeStruct((ne, _H), dt)] * 2,
        scratch_types=([pltpu.VMEM((per_w,), jnp.int32)] * 2
                       + [pltpu.VMEM((_SCK, _H), dt)] * 4
                       + [pltpu.SemaphoreType.DMA] * 8),
    )
    def k(h_hbm, src_hbm, dst_hbm, hs_hbm, hd_hbm,
          idxs, idxd, ra0, rb0, ra1, rb1,
          ga0, gb0, ga1, gb1, wa0, wb0, wa1, wb1):
        wid = lax.axis_index("c") * _NS + lax.axis_index("s")
        base = wid * per_w
        pltpu.sync_copy(src_hbm.at[pl.ds(base, per_w)], idxs)
        pltpu.sync_copy(dst_hbm.at[pl.ds(base, per_w)], idxd)
        bufs = ((ra0, rb0, ga0, gb0, wa0, wb0),
                (ra1, rb1, ga1, gb1, wa1, wb1))

        def gstart(c, b):
            ra, rb, ga, gb, _, _ = bufs[b]
            off = c * _SCK
            pltpu.make_async_copy(
                h_hbm.at[idxs.at[pl.ds(off, _SCK)]], ra, ga).start()
            pltpu.make_async_copy(
                h_hbm.at[idxd.at[pl.ds(off, _SCK)]], rb, gb).start()

        def gfinish(c, b):
            ra, rb, ga, gb, wa, wb = bufs[b]
            off = c * _SCK
            pltpu.make_async_copy(
                h_hbm.at[idxs.at[pl.ds(off, _SCK)]], ra, ga).wait()
            pltpu.make_async_copy(
                h_hbm.at[idxd.at[pl.ds(off, _SCK)]], rb, gb).wait()
            pltpu.make_async_copy(
                ra, hs_hbm.at[pl.ds(base + off, _SCK)], wa).start()
            pltpu.make_async_copy(
                rb, hd_hbm.at[pl.ds(base + off, _SCK)], wb).start()

        def wdrain(c, b):
            ra, rb, _, _, wa, wb = bufs[b]
            off = c * _SCK
            pltpu.make_async_copy(
                ra, hs_hbm.at[pl.ds(base + off, _SCK)], wa).wait()
            pltpu.make_async_copy(
                rb, hd_hbm.at[pl.ds(base + off, _SCK)], wb).wait()

        def body(i, carry):
            c0 = 2 * i
            gstart(c0, 0)
            gstart(c0 + 1, 1)
            gfinish(c0, 0)
            gfinish(c0 + 1, 1)
            wdrain(c0, 0)
            wdrain(c0 + 1, 1)
            return carry

        lax.fori_loop(0, nch // 2, body, 0)
        if nch % 2:
            gstart(nch - 1, 0)
            gfinish(nch - 1, 0)
            wdrain(nch - 1, 0)

    return k(h, src, dst)


def _sc_scatter(eup, dst, n):
    ne = eup.shape[0]
    nw = _NC * _NS
    per_w = ne // nw
    nch = per_w // _SCK
    rows_sub = (n // _NS) // 8 * 8
    rows_last = n - rows_sub * (_NS - 1)
    mesh = plsc.VectorSubcoreMesh(core_axis_name="c", subcore_axis_name="s")

    @functools.partial(
        pl.kernel, mesh=mesh,
        compiler_params=pltpu.CompilerParams(use_tc_tiling_on_sc=False),
        out_type=jax.ShapeDtypeStruct((_NC, n, _H), jnp.float32),
        scratch_types=([pltpu.VMEM_SHARED((n, _H), jnp.float32)]
                       + [pltpu.VMEM((_SCK, _H), jnp.float32)] * 2
                       + [pltpu.VMEM((_SCK,), jnp.int32)] * 2
                       + [pltpu.VMEM((48, _H), jnp.float32)]
                       + [pltpu.SemaphoreType.DMA] * 4),
    )
    def k(eup_hbm, dst_hbm, out_hbm, acc, r0, r1, i0, i1, zbuf, s0, s1, s2, s3):
        cid = lax.axis_index("c")
        sid = lax.axis_index("s")
        base = (cid * _NS + sid) * per_w
        row0 = sid * rows_sub

        def zb(j, carry):
            zbuf[j // 8, pl.ds((j % 8) * 16, 16)] = jnp.zeros((16,), jnp.float32)
            return carry

        lax.fori_loop(0, 48 * 8, zb, 0)
        for z in range(rows_sub // 48):
            pltpu.sync_copy(zbuf, acc.at[pl.ds(row0 + z * 48, 48)])

        @pl.when(sid == _NS - 1)
        def _():
            extra = rows_last - rows_sub
            for z in range(extra // 16):
                pltpu.sync_copy(zbuf.at[pl.ds(0, 16)],
                                acc.at[pl.ds(rows_sub * _NS + z * 16, 16)])

        plsc.subcore_barrier()

        bufs = ((r0, i0, s0, s1), (r1, i1, s2, s3))

        def lstart(c, b):
            r, idx, sr, si = bufs[b]
            off = base + c * _SCK
            pltpu.make_async_copy(eup_hbm.at[pl.ds(off, _SCK)], r, sr).start()
            pltpu.make_async_copy(dst_hbm.at[pl.ds(off, _SCK)], idx, si).start()

        def lfin(c, b):
            r, idx, sr, si = bufs[b]
            off = base + c * _SCK
            pltpu.make_async_copy(eup_hbm.at[pl.ds(off, _SCK)], r, sr).wait()
            pltpu.make_async_copy(dst_hbm.at[pl.ds(off, _SCK)], idx, si).wait()
            pltpu.sync_copy(r, acc.at[idx], add=True)

        def body(i, carry):
            c0 = 2 * i
            lstart(c0, 0)
            lstart(c0 + 1, 1)
            lfin(c0, 0)
            lfin(c0 + 1, 1)
            return carry

        lax.fori_loop(0, nch // 2, body, 0)
        if nch % 2:
            lstart(nch - 1, 0)
            lfin(nch - 1, 0)

        plsc.subcore_barrier()
        pltpu.sync_copy(acc.at[pl.ds(row0, rows_sub)],
                        out_hbm.at[cid, pl.ds(row0, rows_sub)])

        @pl.when(sid == _NS - 1)
        def _():
            extra = rows_last - rows_sub
            pltpu.sync_copy(acc.at[pl.ds(rows_sub * _NS, extra)],
                            out_hbm.at[cid, pl.ds(rows_sub * _NS, extra)])

    return k(eup, dst)


def kernel(x, edge_index, edge_attr,
           enc_n_W1, enc_n_b1, enc_n_W2, enc_n_b2,
           enc_e_W1, enc_e_b1, enc_e_W2, enc_e_b2,
           blk_eW1, blk_eb1, blk_eW2, blk_eb2,
           blk_nW1, blk_nb1, blk_nW2, blk_nb2,
           dec_W1, dec_b1, dec_W2, dec_b2):
    n = x.shape[0]
    ne = edge_attr.shape[0]
    nb = blk_eW1.shape[0]
    bn = 1000 if n % 1000 == 0 else n
    be = 2000 if ne % 2000 == 0 else ne
    src = edge_index[0]
    dst = edge_index[1]
    r = lambda b: b.reshape(1, -1)

    h, hb = _encode_nodes(x, enc_n_W1, r(enc_n_b1), enc_n_W2, r(enc_n_b2), bn)
    e = _encode_edges(edge_attr, enc_e_W1, r(enc_e_b1),
                      enc_e_W2, r(enc_e_b2), be)

    use_sc = (ne % (_NC * _NS * _SCK) == 0) and (n % (_NS * 5) == 0)
    for i in range(nb):
        if use_sc:
            hs, hd = _sc_gather(hb, src, dst)
        else:
            hs, hd = hb[src], hb[dst]
        eup, e = _edge_block(e, hs, hd, blk_eW1[i], r(blk_eb1[i]),
                             blk_eW2[i], r(blk_eb2[i]), be)
        agg = jax.ops.segment_sum(eup, dst, num_segments=n)
        h, hb = _node_block(h, agg, blk_nW1[i], r(blk_nb1[i]),
                            blk_nW2[i], r(blk_nb2[i]), bn)

    w2p = jnp.zeros((_H, _H), jnp.float32).at[:, 0].set(dec_W2[:, 0])
    b2p = jnp.full((1, _H), dec_b2[0], jnp.float32)
    out = _decode(h, dec_W1, r(dec_b1), w2p, b2p, bn)
    return out[:, 0]


# speedup vs baseline: 1.1685x; 1.1685x over previous
import functools

import jax
import jax.numpy as jnp
from jax import lax
from jax.experimental import pallas as pl
from jax.experimental.pallas import tpu as pltpu
from jax.experimental.pallas import tpu_sc as plsc

_H = 128


def _silu(x):
    return x * jax.nn.sigmoid(x)


def _rows(b):
    return pl.BlockSpec((b, _H), lambda i: (i, 0))


def _bcast(shape):
    return pl.BlockSpec(shape, lambda i: (0,) * len(shape))


def _dot(a, b):
    return jnp.dot(a, b, preferred_element_type=jnp.float32)


def _enc_node_body(x_ref, w1, b1, w2, b2, h_ref):
    u = _silu(_dot(x_ref[:], w1[:]) + b1[:])
    h_ref[:] = _dot(u, w2[:]) + b2[:]


def _encode_nodes(x, w1, b1, w2, b2, bn):
    n, nin = x.shape
    return pl.pallas_call(
        _enc_node_body,
        grid=(n // bn,),
        in_specs=[pl.BlockSpec((bn, nin), lambda i: (i, 0)),
                  _bcast((nin, _H)), _bcast((1, _H)),
                  _bcast((_H, _H)), _bcast((1, _H))],
        out_specs=_rows(bn),
        out_shape=jax.ShapeDtypeStruct((n, _H), jnp.float32),
    )(x, w1, b1, w2, b2)


def _enc_edge_body(a_ref, w1, b1, w2, b2, e_ref):
    u = _silu(_dot(a_ref[:], w1[:]) + b1[:])
    e_ref[:] = _dot(u, w2[:]) + b2[:]


def _encode_edges(a, w1, b1, w2, b2, be):
    e, ein = a.shape
    return pl.pallas_call(
        _enc_edge_body,
        grid=(e // be,),
        in_specs=[pl.BlockSpec((be, ein), lambda i: (i, 0)),
                  _bcast((ein, _H)), _bcast((1, _H)),
                  _bcast((_H, _H)), _bcast((1, _H))],
        out_specs=_rows(be),
        out_shape=jax.ShapeDtypeStruct((e, _H), jnp.float32),
    )(a, w1, b1, w2, b2)


def _edge_blk_body(e_ref, hs_ref, hd_ref, w1, b1, w2, b2, eup_ref, enew_ref):
    e = e_ref[:]
    e_in = jnp.concatenate([e, hs_ref[:], hd_ref[:]], axis=-1)
    u = _silu(_dot(e_in, w1[:]) + b1[:])
    eup = _dot(u, w2[:]) + b2[:]
    eup_ref[:] = eup
    enew_ref[:] = e + eup


def _edge_block(e, hs, hd, w1, b1, w2, b2, be):
    ne = e.shape[0]
    return pl.pallas_call(
        _edge_blk_body,
        grid=(ne // be,),
        in_specs=[_rows(be), _rows(be), _rows(be),
                  _bcast((3 * _H, _H)), _bcast((1, _H)),
                  _bcast((_H, _H)), _bcast((1, _H))],
        out_specs=[_rows(be), _rows(be)],
        out_shape=[jax.ShapeDtypeStruct((ne, _H), jnp.float32)] * 2,
    )(e, hs, hd, w1, b1, w2, b2)


def _node_blk_body(h_ref, agg_ref, w1, b1, w2, b2, hn_ref):
    h = h_ref[:]
    x_in = jnp.concatenate([h, agg_ref[:]], axis=-1)
    u = _silu(_dot(x_in, w1[:]) + b1[:])
    hn_ref[:] = h + _dot(u, w2[:]) + b2[:]


def _node_block(h, agg, w1, b1, w2, b2, bn):
    n = h.shape[0]
    return pl.pallas_call(
        _node_blk_body,
        grid=(n // bn,),
        in_specs=[_rows(bn), _rows(bn),
                  _bcast((2 * _H, _H)), _bcast((1, _H)),
                  _bcast((_H, _H)), _bcast((1, _H))],
        out_specs=_rows(bn),
        out_shape=jax.ShapeDtypeStruct((n, _H), jnp.float32),
    )(h, agg, w1, b1, w2, b2)


def _dec_body(h_ref, w1, b1, w2, b2, o_ref):
    u = _silu(_dot(h_ref[:], w1[:]) + b1[:])
    o_ref[:] = _dot(u, w2[:]) + b2[:]


def _decode(h, w1, b1, w2p, b2p, bn):
    n = h.shape[0]
    return pl.pallas_call(
        _dec_body,
        grid=(n // bn,),
        in_specs=[_rows(bn),
                  _bcast((_H, _H)), _bcast((1, _H)),
                  _bcast((_H, _H)), _bcast((1, _H))],
        out_specs=_rows(bn),
        out_shape=jax.ShapeDtypeStruct((n, _H), jnp.float32),
    )(h, w1, b1, w2p, b2p)


_SCK = 80
_NC = 2
_NS = 16


def _sc_gather(h, src, dst):
    ne = src.shape[0]
    nw = _NC * _NS
    per_w = ne // nw
    nch = per_w // _SCK
    dt = h.dtype
    mesh = plsc.VectorSubcoreMesh(core_axis_name="c", subcore_axis_name="s")

    @functools.partial(
        pl.kernel, mesh=mesh,
        compiler_params=pltpu.CompilerParams(use_tc_tiling_on_sc=False),
        out_type=[jax.ShapeDtypeStruct((ne, _H), dt)] * 2,
        scratch_types=([pltpu.VMEM((per_w,), jnp.int32)] * 2
                       + [pltpu.VMEM((_SCK, _H), dt)] * 4
                       + [pltpu.SemaphoreType.DMA] * 8),
    )
    def k(h_hbm, src_hbm, dst_hbm, hs_hbm, hd_hbm,
          idxs, idxd, ra0, rb0, ra1, rb1,
          ga0, gb0, ga1, gb1, wa0, wb0, wa1, wb1):
        wid = lax.axis_index("c") * _NS + lax.axis_index("s")
        base = wid * per_w
        pltpu.sync_copy(src_hbm.at[pl.ds(base, per_w)], idxs)
        pltpu.sync_copy(dst_hbm.at[pl.ds(base, per_w)], idxd)
        bufs = ((ra0, rb0, ga0, gb0, wa0, wb0),
                (ra1, rb1, ga1, gb1, wa1, wb1))

        def gstart(c, b):
            ra, rb, ga, gb, _, _ = bufs[b]
            off = c * _SCK
            pltpu.make_async_copy(
                h_hbm.at[idxs.at[pl.ds(off, _SCK)]], ra, ga).start()
            pltpu.make_async_copy(
                h_hbm.at[idxd.at[pl.ds(off, _SCK)]], rb, gb).start()

        def gfinish(c, b):
            ra, rb, ga, gb, wa, wb = bufs[b]
            off = c * _SCK
            pltpu.make_async_copy(
                h_hbm.at[idxs.at[pl.ds(off, _SCK)]], ra, ga).wait()
            pltpu.make_async_copy(
                h_hbm.at[idxd.at[pl.ds(off, _SCK)]], rb, gb).wait()
            pltpu.make_async_copy(
                ra, hs_hbm.at[pl.ds(base + off, _SCK)], wa).start()
            pltpu.make_async_copy(
                rb, hd_hbm.at[pl.ds(base + off, _SCK)], wb).start()

        def wdrain(c, b):
            ra, rb, _, _, wa, wb = bufs[b]
            off = c * _SCK
            pltpu.make_async_copy(
                ra, hs_hbm.at[pl.ds(base + off, _SCK)], wa).wait()
            pltpu.make_async_copy(
                rb, hd_hbm.at[pl.ds(base + off, _SCK)], wb).wait()

        def body(i, carry):
            c0 = 2 * i
            gstart(c0, 0)
            gstart(c0 + 1, 1)
            gfinish(c0, 0)
            gfinish(c0 + 1, 1)
            wdrain(c0, 0)
            wdrain(c0 + 1, 1)
            return carry

        lax.fori_loop(0, nch // 2, body, 0)
        if nch % 2:
            gstart(nch - 1, 0)
            gfinish(nch - 1, 0)
            wdrain(nch - 1, 0)

    return k(h, src, dst)


def _sc_scatter(eup, dst, n):
    ne = eup.shape[0]
    nw = _NC * _NS
    per_w = ne // nw
    nch = per_w // _SCK
    rows_sub = (n // _NS) // 8 * 8
    rows_last = n - rows_sub * (_NS - 1)
    mesh = plsc.VectorSubcoreMesh(core_axis_name="c", subcore_axis_name="s")

    @functools.partial(
        pl.kernel, mesh=mesh,
        compiler_params=pltpu.CompilerParams(use_tc_tiling_on_sc=False),
        out_type=jax.ShapeDtypeStruct((_NC, n, _H), jnp.float32),
        scratch_types=([pltpu.VMEM_SHARED((n, _H), jnp.float32)]
                       + [pltpu.VMEM((_SCK, _H), jnp.float32)] * 2
                       + [pltpu.VMEM((_SCK,), jnp.int32)] * 2
                       + [pltpu.VMEM((48, _H), jnp.float32)]
                       + [pltpu.SemaphoreType.DMA] * 4),
    )
    def k(eup_hbm, dst_hbm, out_hbm, acc, r0, r1, i0, i1, zbuf, s0, s1, s2, s3):
        cid = lax.axis_index("c")
        sid = lax.axis_index("s")
        base = (cid * _NS + sid) * per_w
        row0 = sid * rows_sub

        def zb(j, carry):
            zbuf[j // 8, pl.ds((j % 8) * 16, 16)] = jnp.zeros((16,), jnp.float32)
            return carry

        lax.fori_loop(0, 48 * 8, zb, 0)
        for z in range(rows_sub // 48):
            pltpu.sync_copy(zbuf, acc.at[pl.ds(row0 + z * 48, 48)])

        @pl.when(sid == _NS - 1)
        def _():
            extra = rows_last - rows_sub
            for z in range(extra // 16):
                pltpu.sync_copy(zbuf.at[pl.ds(0, 16)],
                                acc.at[pl.ds(rows_sub * _NS + z * 16, 16)])

        plsc.subcore_barrier()

        bufs = ((r0, i0, s0, s1), (r1, i1, s2, s3))

        def lstart(c, b):
            r, idx, sr, si = bufs[b]
            off = base + c * _SCK
            pltpu.make_async_copy(eup_hbm.at[pl.ds(off, _SCK)], r, sr).start()
            pltpu.make_async_copy(dst_hbm.at[pl.ds(off, _SCK)], idx, si).start()

        def lfin(c, b):
            r, idx, sr, si = bufs[b]
            off = base + c * _SCK
            pltpu.make_async_copy(eup_hbm.at[pl.ds(off, _SCK)], r, sr).wait()
            pltpu.make_async_copy(dst_hbm.at[pl.ds(off, _SCK)], idx, si).wait()
            pltpu.sync_copy(r, acc.at[idx], add=True)

        def body(i, carry):
            c0 = 2 * i
            lstart(c0, 0)
            lstart(c0 + 1, 1)
            lfin(c0, 0)
            lfin(c0 + 1, 1)
            return carry

        lax.fori_loop(0, nch // 2, body, 0)
        if nch % 2:
            lstart(nch - 1, 0)
            lfin(nch - 1, 0)

        plsc.subcore_barrier()
        pltpu.sync_copy(acc.at[pl.ds(row0, rows_sub)],
                        out_hbm.at[cid, pl.ds(row0, rows_sub)])

        @pl.when(sid == _NS - 1)
        def _():
            extra = rows_last - rows_sub
            pltpu.sync_copy(acc.at[pl.ds(rows_sub * _NS, extra)],
                            out_hbm.at[cid, pl.ds(rows_sub * _NS, extra)])

    return k(eup, dst)


def kernel(x, edge_index, edge_attr,
           enc_n_W1, enc_n_b1, enc_n_W2, enc_n_b2,
           enc_e_W1, enc_e_b1, enc_e_W2, enc_e_b2,
           blk_eW1, blk_eb1, blk_eW2, blk_eb2,
           blk_nW1, blk_nb1, blk_nW2, blk_nb2,
           dec_W1, dec_b1, dec_W2, dec_b2):
    n = x.shape[0]
    ne = edge_attr.shape[0]
    nb = blk_eW1.shape[0]
    bn = 1000 if n % 1000 == 0 else n
    be = 2000 if ne % 2000 == 0 else ne
    src = edge_index[0]
    dst = edge_index[1]
    order = jnp.argsort(dst, stable=True)
    src = src[order]
    dst = dst[order]
    edge_attr = edge_attr[order]
    r = lambda b: b.reshape(1, -1)

    h = _encode_nodes(x, enc_n_W1, r(enc_n_b1), enc_n_W2, r(enc_n_b2), bn)
    e = _encode_edges(edge_attr, enc_e_W1, r(enc_e_b1),
                      enc_e_W2, r(enc_e_b2), be)

    use_sc = (ne % (_NC * _NS * _SCK) == 0) and (n % (_NS * 5) == 0)
    for i in range(nb):
        if use_sc:
            hs, hd = _sc_gather(h, src, dst)
        else:
            hs, hd = h[src], h[dst]
        eup, e = _edge_block(e, hs, hd, blk_eW1[i], r(blk_eb1[i]),
                             blk_eW2[i], r(blk_eb2[i]), be)
        agg = jax.ops.segment_sum(eup, dst, num_segments=n,
                                  indices_are_sorted=True)
        h = _node_block(h, agg, blk_nW1[i], r(blk_nb1[i]),
                        blk_nW2[i], r(blk_nb2[i]), bn)

    w2p = jnp.zeros((_H, _H), jnp.float32).at[:, 0].set(dec_W2[:, 0])
    b2p = jnp.full((1, _H), dec_b2[0], jnp.float32)
    out = _decode(h, dec_W1, r(dec_b1), w2p, b2p, bn)
    return out[:, 0]


# speedup vs baseline: 1.4408x; 1.2331x over previous
import functools

import jax
import jax.numpy as jnp
from jax import lax
from jax.experimental import pallas as pl
from jax.experimental.pallas import tpu as pltpu
from jax.experimental.pallas import tpu_sc as plsc

_H = 128


def _silu(x):
    return x * jax.nn.sigmoid(x)


def _rows(b):
    return pl.BlockSpec((b, _H), lambda i: (i, 0))


def _bcast(shape):
    return pl.BlockSpec(shape, lambda i: (0,) * len(shape))


def _dot(a, b):
    return jnp.dot(a, b, preferred_element_type=jnp.float32)


def _enc_node_body(x_ref, w1, b1, w2, b2, h_ref):
    u = _silu(_dot(x_ref[:], w1[:]) + b1[:])
    h_ref[:] = _dot(u, w2[:]) + b2[:]


def _encode_nodes(x, w1, b1, w2, b2, bn):
    n, nin = x.shape
    return pl.pallas_call(
        _enc_node_body,
        grid=(n // bn,),
        in_specs=[pl.BlockSpec((bn, nin), lambda i: (i, 0)),
                  _bcast((nin, _H)), _bcast((1, _H)),
                  _bcast((_H, _H)), _bcast((1, _H))],
        out_specs=_rows(bn),
        out_shape=jax.ShapeDtypeStruct((n, _H), jnp.float32),
    )(x, w1, b1, w2, b2)


def _enc_edge_body(a_ref, w1, b1, w2, b2, e_ref):
    u = _silu(_dot(a_ref[:], w1[:]) + b1[:])
    e_ref[:] = _dot(u, w2[:]) + b2[:]


def _encode_edges(a, w1, b1, w2, b2, be):
    e, ein = a.shape
    return pl.pallas_call(
        _enc_edge_body,
        grid=(e // be,),
        in_specs=[pl.BlockSpec((be, ein), lambda i: (i, 0)),
                  _bcast((ein, _H)), _bcast((1, _H)),
                  _bcast((_H, _H)), _bcast((1, _H))],
        out_specs=_rows(be),
        out_shape=jax.ShapeDtypeStruct((e, _H), jnp.float32),
    )(a, w1, b1, w2, b2)


def _edge_blk_body(e_ref, hs_ref, hd_ref, w1, b1, w2, b2, eup_ref, enew_ref):
    e = e_ref[:]
    e_in = jnp.concatenate([e, hs_ref[:], hd_ref[:]], axis=-1)
    u = _silu(_dot(e_in, w1[:]) + b1[:])
    eup = _dot(u, w2[:]) + b2[:]
    eup_ref[:] = eup
    enew_ref[:] = e + eup


def _edge_block(e, hs, hd, w1, b1, w2, b2, be):
    ne = e.shape[0]
    return pl.pallas_call(
        _edge_blk_body,
        grid=(ne // be,),
        in_specs=[_rows(be), _rows(be), _rows(be),
                  _bcast((3 * _H, _H)), _bcast((1, _H)),
                  _bcast((_H, _H)), _bcast((1, _H))],
        out_specs=[_rows(be), _rows(be)],
        out_shape=[jax.ShapeDtypeStruct((ne, _H), jnp.float32)] * 2,
    )(e, hs, hd, w1, b1, w2, b2)


def _node_blk_body(h_ref, agg_ref, w1, b1, w2, b2, hn_ref):
    h = h_ref[:]
    x_in = jnp.concatenate([h, agg_ref[:]], axis=-1)
    u = _silu(_dot(x_in, w1[:]) + b1[:])
    hn_ref[:] = h + _dot(u, w2[:]) + b2[:]


def _node_block(h, agg, w1, b1, w2, b2, bn):
    n = h.shape[0]
    return pl.pallas_call(
        _node_blk_body,
        grid=(n // bn,),
        in_specs=[_rows(bn), _rows(bn),
                  _bcast((2 * _H, _H)), _bcast((1, _H)),
                  _bcast((_H, _H)), _bcast((1, _H))],
        out_specs=_rows(bn),
        out_shape=jax.ShapeDtypeStruct((n, _H), jnp.float32),
    )(h, agg, w1, b1, w2, b2)


def _dec_body(h_ref, w1, b1, w2, b2, o_ref):
    u = _silu(_dot(h_ref[:], w1[:]) + b1[:])
    o_ref[:] = _dot(u, w2[:]) + b2[:]


def _decode(h, w1, b1, w2p, b2p, bn):
    n = h.shape[0]
    return pl.pallas_call(
        _dec_body,
        grid=(n // bn,),
        in_specs=[_rows(bn),
                  _bcast((_H, _H)), _bcast((1, _H)),
                  _bcast((_H, _H)), _bcast((1, _H))],
        out_specs=_rows(bn),
        out_shape=jax.ShapeDtypeStruct((n, _H), jnp.float32),
    )(h, w1, b1, w2p, b2p)


_SCK = 80
_NC = 2
_NS = 16


def _sc_gather(h, src, dst):
    ne = src.shape[0]
    nw = _NC * _NS
    per_w = ne // nw
    nch = per_w // _SCK
    dt = h.dtype
    mesh = plsc.VectorSubcoreMesh(core_axis_name="c", subcore_axis_name="s")

    @functools.partial(
        pl.kernel, mesh=mesh,
        compiler_params=pltpu.CompilerParams(use_tc_tiling_on_sc=False),
        out_type=[jax.ShapeDtypeStruct((ne, _H), dt)] * 2,
        scratch_types=([pltpu.VMEM((per_w,), jnp.int32)] * 2
                       + [pltpu.VMEM((_SCK, _H), dt)] * 8
                       + [pltpu.SemaphoreType.DMA] * 16),
    )
    def k(h_hbm, src_hbm, dst_hbm, hs_hbm, hd_hbm,
          idxs, idxd, ra0, rb0, ra1, rb1, ra2, rb2, ra3, rb3,
          ga0, gb0, ga1, gb1, ga2, gb2, ga3, gb3,
          wa0, wb0, wa1, wb1, wa2, wb2, wa3, wb3):
        wid = lax.axis_index("c") * _NS + lax.axis_index("s")
        base = wid * per_w
        pltpu.sync_copy(src_hbm.at[pl.ds(base, per_w)], idxs)
        pltpu.sync_copy(dst_hbm.at[pl.ds(base, per_w)], idxd)
        bufs = ((ra0, rb0, ga0, gb0, wa0, wb0),
                (ra1, rb1, ga1, gb1, wa1, wb1),
                (ra2, rb2, ga2, gb2, wa2, wb2),
                (ra3, rb3, ga3, gb3, wa3, wb3))

        def gstart(c, b):
            ra, rb, ga, gb, _, _ = bufs[b]
            off = c * _SCK
            pltpu.make_async_copy(
                h_hbm.at[idxs.at[pl.ds(off, _SCK)]], ra, ga).start()
            pltpu.make_async_copy(
                h_hbm.at[idxd.at[pl.ds(off, _SCK)]], rb, gb).start()

        def gfinish(c, b):
            ra, rb, ga, gb, wa, wb = bufs[b]
            off = c * _SCK
            pltpu.make_async_copy(
                h_hbm.at[idxs.at[pl.ds(off, _SCK)]], ra, ga).wait()
            pltpu.make_async_copy(
                h_hbm.at[idxd.at[pl.ds(off, _SCK)]], rb, gb).wait()
            pltpu.make_async_copy(
                ra, hs_hbm.at[pl.ds(base + off, _SCK)], wa).start()
            pltpu.make_async_copy(
                rb, hd_hbm.at[pl.ds(base + off, _SCK)], wb).start()

        def wdrain(c, b):
            ra, rb, _, _, wa, wb = bufs[b]
            off = c * _SCK
            pltpu.make_async_copy(
                ra, hs_hbm.at[pl.ds(base + off, _SCK)], wa).wait()
            pltpu.make_async_copy(
                rb, hd_hbm.at[pl.ds(base + off, _SCK)], wb).wait()

        nq = nch // 4
        for b in range(4):
            gstart(b, b)

        def body(i, carry):
            c0 = 4 * i
            for b in range(4):
                gfinish(c0 + b, b)
            for b in range(4):
                wdrain(c0 + b, b)
                nxt = c0 + 4 + b

                @pl.when(nxt < nch)
                def _(nxt=nxt, b=b):
                    gstart(nxt, b)
            return carry

        lax.fori_loop(0, nq, body, 0)
        for b in range(nch % 4):
            c = nq * 4 + b
            gfinish(c, b)
            wdrain(c, b)

    return k(h, src, dst)


def _sc_scatter(eup, dst, n):
    ne = eup.shape[0]
    nw = _NC * _NS
    per_w = ne // nw
    nch = per_w // _SCK
    rows_sub = (n // _NS) // 8 * 8
    rows_last = n - rows_sub * (_NS - 1)
    mesh = plsc.VectorSubcoreMesh(core_axis_name="c", subcore_axis_name="s")

    @functools.partial(
        pl.kernel, mesh=mesh,
        compiler_params=pltpu.CompilerParams(use_tc_tiling_on_sc=False),
        out_type=jax.ShapeDtypeStruct((_NC, n, _H), jnp.float32),
        scratch_types=([pltpu.VMEM_SHARED((n, _H), jnp.float32)]
                       + [pltpu.VMEM((_SCK, _H), jnp.float32)] * 2
                       + [pltpu.VMEM((_SCK,), jnp.int32)] * 2
                       + [pltpu.VMEM((48, _H), jnp.float32)]
                       + [pltpu.SemaphoreType.DMA] * 4),
    )
    def k(eup_hbm, dst_hbm, out_hbm, acc, r0, r1, i0, i1, zbuf, s0, s1, s2, s3):
        cid = lax.axis_index("c")
        sid = lax.axis_index("s")
        base = (cid * _NS + sid) * per_w
        row0 = sid * rows_sub

        def zb(j, carry):
            zbuf[j // 8, pl.ds((j % 8) * 16, 16)] = jnp.zeros((16,), jnp.float32)
            return carry

        lax.fori_loop(0, 48 * 8, zb, 0)
        for z in range(rows_sub // 48):
            pltpu.sync_copy(zbuf, acc.at[pl.ds(row0 + z * 48, 48)])

        @pl.when(sid == _NS - 1)
        def _():
            extra = rows_last - rows_sub
            for z in range(extra // 16):
                pltpu.sync_copy(zbuf.at[pl.ds(0, 16)],
                                acc.at[pl.ds(rows_sub * _NS + z * 16, 16)])

        plsc.subcore_barrier()

        bufs = ((r0, i0, s0, s1), (r1, i1, s2, s3))

        def lstart(c, b):
            r, idx, sr, si = bufs[b]
            off = base + c * _SCK
            pltpu.make_async_copy(eup_hbm.at[pl.ds(off, _SCK)], r, sr).start()
            pltpu.make_async_copy(dst_hbm.at[pl.ds(off, _SCK)], idx, si).start()

        def lfin(c, b):
            r, idx, sr, si = bufs[b]
            off = base + c * _SCK
            pltpu.make_async_copy(eup_hbm.at[pl.ds(off, _SCK)], r, sr).wait()
            pltpu.make_async_copy(dst_hbm.at[pl.ds(off, _SCK)], idx, si).wait()
            pltpu.sync_copy(r, acc.at[idx], add=True)

        def body(i, carry):
            c0 = 2 * i
            lstart(c0, 0)
            lstart(c0 + 1, 1)
            lfin(c0, 0)
            lfin(c0 + 1, 1)
            return carry

        lax.fori_loop(0, nch // 2, body, 0)
        if nch % 2:
            lstart(nch - 1, 0)
            lfin(nch - 1, 0)

        plsc.subcore_barrier()
        pltpu.sync_copy(acc.at[pl.ds(row0, rows_sub)],
                        out_hbm.at[cid, pl.ds(row0, rows_sub)])

        @pl.when(sid == _NS - 1)
        def _():
            extra = rows_last - rows_sub
            pltpu.sync_copy(acc.at[pl.ds(rows_sub * _NS, extra)],
                            out_hbm.at[cid, pl.ds(rows_sub * _NS, extra)])

    return k(eup, dst)


def kernel(x, edge_index, edge_attr,
           enc_n_W1, enc_n_b1, enc_n_W2, enc_n_b2,
           enc_e_W1, enc_e_b1, enc_e_W2, enc_e_b2,
           blk_eW1, blk_eb1, blk_eW2, blk_eb2,
           blk_nW1, blk_nb1, blk_nW2, blk_nb2,
           dec_W1, dec_b1, dec_W2, dec_b2):
    n = x.shape[0]
    ne = edge_attr.shape[0]
    nb = blk_eW1.shape[0]
    bn = 1000 if n % 1000 == 0 else n
    be = 2000 if ne % 2000 == 0 else ne
    src = edge_index[0]
    dst = edge_index[1]
    r = lambda b: b.reshape(1, -1)

    h = _encode_nodes(x, enc_n_W1, r(enc_n_b1), enc_n_W2, r(enc_n_b2), bn)
    e = _encode_edges(edge_attr, enc_e_W1, r(enc_e_b1),
                      enc_e_W2, r(enc_e_b2), be)

    use_sc = (ne % (_NC * _NS * _SCK) == 0) and (n % (_NS * 5) == 0)
    for i in range(nb):
        if use_sc:
            hs, hd = _sc_gather(h, src, dst)
        else:
            hs, hd = h[src], h[dst]
        eup, e = _edge_block(e, hs, hd, blk_eW1[i], r(blk_eb1[i]),
                             blk_eW2[i], r(blk_eb2[i]), be)
        agg = jax.ops.segment_sum(eup, dst, num_segments=n)
        h = _node_block(h, agg, blk_nW1[i], r(blk_nb1[i]),
                        blk_nW2[i], r(blk_nb2[i]), bn)

    w2p = jnp.zeros((_H, _H), jnp.float32).at[:, 0].set(dec_W2[:, 0])
    b2p = jnp.full((1, _H), dec_b2[0], jnp.float32)
    out = _decode(h, dec_W1, r(dec_b1), w2p, b2p, bn)
    return out[:, 0]


# speedup vs baseline: 1.4853x; 1.0309x over previous
import functools

import jax
import jax.numpy as jnp
from jax import lax
from jax.experimental import pallas as pl
from jax.experimental.pallas import tpu as pltpu
from jax.experimental.pallas import tpu_sc as plsc

_H = 128


def _silu(x):
    return x * jax.nn.sigmoid(x)


def _rows(b):
    return pl.BlockSpec((b, _H), lambda i: (i, 0))


def _bcast(shape):
    return pl.BlockSpec(shape, lambda i: (0,) * len(shape))


def _dot(a, b):
    return jnp.dot(a, b, preferred_element_type=jnp.float32)


def _enc_node_body(x_ref, w1, b1, w2, b2, h_ref):
    u = _silu(_dot(x_ref[:], w1[:]) + b1[:])
    h_ref[:] = _dot(u, w2[:]) + b2[:]


def _encode_nodes(x, w1, b1, w2, b2, bn):
    n, nin = x.shape
    return pl.pallas_call(
        _enc_node_body,
        grid=(n // bn,),
        in_specs=[pl.BlockSpec((bn, nin), lambda i: (i, 0)),
                  _bcast((nin, _H)), _bcast((1, _H)),
                  _bcast((_H, _H)), _bcast((1, _H))],
        out_specs=_rows(bn),
        out_shape=jax.ShapeDtypeStruct((n, _H), jnp.float32),
    )(x, w1, b1, w2, b2)


def _enc_edge_body(a_ref, w1, b1, w2, b2, e_ref):
    u = _silu(_dot(a_ref[:], w1[:]) + b1[:])
    e_ref[:] = _dot(u, w2[:]) + b2[:]


def _encode_edges(a, w1, b1, w2, b2, be):
    e, ein = a.shape
    return pl.pallas_call(
        _enc_edge_body,
        grid=(e // be,),
        in_specs=[pl.BlockSpec((be, ein), lambda i: (i, 0)),
                  _bcast((ein, _H)), _bcast((1, _H)),
                  _bcast((_H, _H)), _bcast((1, _H))],
        out_specs=_rows(be),
        out_shape=jax.ShapeDtypeStruct((e, _H), jnp.float32),
    )(a, w1, b1, w2, b2)


def _edge_blk_body(e_ref, hs_ref, hd_ref, w1, b1, w2, b2, eup_ref, enew_ref):
    e = e_ref[:]
    e_in = jnp.concatenate([e, hs_ref[:], hd_ref[:]], axis=-1)
    u = _silu(_dot(e_in, w1[:]) + b1[:])
    eup = _dot(u, w2[:]) + b2[:]
    eup_ref[:] = eup
    enew_ref[:] = e + eup


def _edge_block(e, hs, hd, w1, b1, w2, b2, be):
    ne = e.shape[0]
    return pl.pallas_call(
        _edge_blk_body,
        grid=(ne // be,),
        in_specs=[_rows(be), _rows(be), _rows(be),
                  _bcast((3 * _H, _H)), _bcast((1, _H)),
                  _bcast((_H, _H)), _bcast((1, _H))],
        out_specs=[_rows(be), _rows(be)],
        out_shape=[jax.ShapeDtypeStruct((ne, _H), jnp.float32)] * 2,
    )(e, hs, hd, w1, b1, w2, b2)


def _node_blk_body(h_ref, agg_ref, w1, b1, w2, b2, hn_ref):
    h = h_ref[:]
    x_in = jnp.concatenate([h, agg_ref[:]], axis=-1)
    u = _silu(_dot(x_in, w1[:]) + b1[:])
    hn_ref[:] = h + _dot(u, w2[:]) + b2[:]


def _node_block(h, agg, w1, b1, w2, b2, bn):
    n = h.shape[0]
    return pl.pallas_call(
        _node_blk_body,
        grid=(n // bn,),
        in_specs=[_rows(bn), _rows(bn),
                  _bcast((2 * _H, _H)), _bcast((1, _H)),
                  _bcast((_H, _H)), _bcast((1, _H))],
        out_specs=_rows(bn),
        out_shape=jax.ShapeDtypeStruct((n, _H), jnp.float32),
    )(h, agg, w1, b1, w2, b2)


def _dec_body(h_ref, w1, b1, w2, b2, o_ref):
    u = _silu(_dot(h_ref[:], w1[:]) + b1[:])
    o_ref[:] = _dot(u, w2[:]) + b2[:]


def _decode(h, w1, b1, w2p, b2p, bn):
    n = h.shape[0]
    return pl.pallas_call(
        _dec_body,
        grid=(n // bn,),
        in_specs=[_rows(bn),
                  _bcast((_H, _H)), _bcast((1, _H)),
                  _bcast((_H, _H)), _bcast((1, _H))],
        out_specs=_rows(bn),
        out_shape=jax.ShapeDtypeStruct((n, _H), jnp.float32),
    )(h, w1, b1, w2p, b2p)


_SCK = 80
_NC = 2
_NS = 16


def _sc_gather(h, src, dst):
    ne = src.shape[0]
    nw = _NC * _NS
    per_w = ne // nw
    nch = per_w // _SCK
    dt = h.dtype
    mesh = plsc.VectorSubcoreMesh(core_axis_name="c", subcore_axis_name="s")

    @functools.partial(
        pl.kernel, mesh=mesh,
        compiler_params=pltpu.CompilerParams(use_tc_tiling_on_sc=False),
        out_type=[jax.ShapeDtypeStruct((ne, _H), dt)] * 2,
        scratch_types=([pltpu.VMEM((per_w,), jnp.int32)] * 2
                       + [pltpu.VMEM((_SCK, _H), dt)] * 8
                       + [pltpu.SemaphoreType.DMA] * 16),
    )
    def k(h_hbm, src_hbm, dst_hbm, hs_hbm, hd_hbm,
          idxs, idxd, ra0, rb0, ra1, rb1, ra2, rb2, ra3, rb3,
          ga0, gb0, ga1, gb1, ga2, gb2, ga3, gb3,
          wa0, wb0, wa1, wb1, wa2, wb2, wa3, wb3):
        wid = lax.axis_index("c") * _NS + lax.axis_index("s")
        base = wid * per_w
        pltpu.sync_copy(src_hbm.at[pl.ds(base, per_w)], idxs)
        pltpu.sync_copy(dst_hbm.at[pl.ds(base, per_w)], idxd)
        bufs = ((ra0, rb0, ga0, gb0, wa0, wb0),
                (ra1, rb1, ga1, gb1, wa1, wb1),
                (ra2, rb2, ga2, gb2, wa2, wb2),
                (ra3, rb3, ga3, gb3, wa3, wb3))

        def gstart(c, b):
            ra, rb, ga, gb, _, _ = bufs[b]
            off = c * _SCK
            pltpu.make_async_copy(
                h_hbm.at[idxs.at[pl.ds(off, _SCK)]], ra, ga).start()
            pltpu.make_async_copy(
                h_hbm.at[idxd.at[pl.ds(off, _SCK)]], rb, gb).start()

        def gfinish(c, b):
            ra, rb, ga, gb, wa, wb = bufs[b]
            off = c * _SCK
            pltpu.make_async_copy(
                h_hbm.at[idxs.at[pl.ds(off, _SCK)]], ra, ga).wait()
            pltpu.make_async_copy(
                h_hbm.at[idxd.at[pl.ds(off, _SCK)]], rb, gb).wait()
            pltpu.make_async_copy(
                ra, hs_hbm.at[pl.ds(base + off, _SCK)], wa).start()
            pltpu.make_async_copy(
                rb, hd_hbm.at[pl.ds(base + off, _SCK)], wb).start()

        def wdrain(c, b):
            ra, rb, _, _, wa, wb = bufs[b]
            off = c * _SCK
            pltpu.make_async_copy(
                ra, hs_hbm.at[pl.ds(base + off, _SCK)], wa).wait()
            pltpu.make_async_copy(
                rb, hd_hbm.at[pl.ds(base + off, _SCK)], wb).wait()

        nq = nch // 4
        for b in range(4):
            gstart(b, b)

        def body(i, carry):
            c0 = 4 * i
            for b in range(4):
                gfinish(c0 + b, b)
            for b in range(4):
                wdrain(c0 + b, b)
                nxt = c0 + 4 + b

                @pl.when(nxt < nch)
                def _(nxt=nxt, b=b):
                    gstart(nxt, b)
            return carry

        lax.fori_loop(0, nq, body, 0)
        for b in range(nch % 4):
            c = nq * 4 + b
            gfinish(c, b)
            wdrain(c, b)

    return k(h, src, dst)


def _sc_scatter(eup, dst, n):
    ne = eup.shape[0]
    nw = _NC * _NS
    per_w = ne // nw
    nch = per_w // _SCK
    rows_sub = (n // _NS) // 8 * 8
    rows_last = n - rows_sub * (_NS - 1)
    mesh = plsc.VectorSubcoreMesh(core_axis_name="c", subcore_axis_name="s")

    @functools.partial(
        pl.kernel, mesh=mesh,
        compiler_params=pltpu.CompilerParams(use_tc_tiling_on_sc=False),
        out_type=jax.ShapeDtypeStruct((_NC, n, _H), jnp.float32),
        scratch_types=([pltpu.VMEM_SHARED((n, _H), jnp.float32)]
                       + [pltpu.VMEM((_SCK, _H), jnp.float32)] * 2
                       + [pltpu.VMEM((_SCK,), jnp.int32)] * 2
                       + [pltpu.VMEM((48, _H), jnp.float32)]
                       + [pltpu.SemaphoreType.DMA] * 4),
    )
    def k(eup_hbm, dst_hbm, out_hbm, acc, r0, r1, i0, i1, zbuf, s0, s1, s2, s3):
        cid = lax.axis_index("c")
        sid = lax.axis_index("s")
        base = (cid * _NS + sid) * per_w
        row0 = sid * rows_sub

        def zb(j, carry):
            zbuf[j // 8, pl.ds((j % 8) * 16, 16)] = jnp.zeros((16,), jnp.float32)
            return carry

        lax.fori_loop(0, 48 * 8, zb, 0)
        for z in range(rows_sub // 48):
            pltpu.sync_copy(zbuf, acc.at[pl.ds(row0 + z * 48, 48)])

        @pl.when(sid == _NS - 1)
        def _():
            extra = rows_last - rows_sub
            for z in range(extra // 16):
                pltpu.sync_copy(zbuf.at[pl.ds(0, 16)],
                                acc.at[pl.ds(rows_sub * _NS + z * 16, 16)])

        plsc.subcore_barrier()

        bufs = ((r0, i0, s0, s1), (r1, i1, s2, s3))

        def lstart(c, b):
            r, idx, sr, si = bufs[b]
            off = base + c * _SCK
            pltpu.make_async_copy(eup_hbm.at[pl.ds(off, _SCK)], r, sr).start()
            pltpu.make_async_copy(dst_hbm.at[pl.ds(off, _SCK)], idx, si).start()

        def lfin(c, b):
            r, idx, sr, si = bufs[b]
            off = base + c * _SCK
            pltpu.make_async_copy(eup_hbm.at[pl.ds(off, _SCK)], r, sr).wait()
            pltpu.make_async_copy(dst_hbm.at[pl.ds(off, _SCK)], idx, si).wait()
            pltpu.sync_copy(r, acc.at[idx], add=True)

        def body(i, carry):
            c0 = 2 * i
            lstart(c0, 0)
            lstart(c0 + 1, 1)
            lfin(c0, 0)
            lfin(c0 + 1, 1)
            return carry

        lax.fori_loop(0, nch // 2, body, 0)
        if nch % 2:
            lstart(nch - 1, 0)
            lfin(nch - 1, 0)

        plsc.subcore_barrier()
        pltpu.sync_copy(acc.at[pl.ds(row0, rows_sub)],
                        out_hbm.at[cid, pl.ds(row0, rows_sub)])

        @pl.when(sid == _NS - 1)
        def _():
            extra = rows_last - rows_sub
            pltpu.sync_copy(acc.at[pl.ds(rows_sub * _NS, extra)],
                            out_hbm.at[cid, pl.ds(rows_sub * _NS, extra)])

    return k(eup, dst)


def kernel(x, edge_index, edge_attr,
           enc_n_W1, enc_n_b1, enc_n_W2, enc_n_b2,
           enc_e_W1, enc_e_b1, enc_e_W2, enc_e_b2,
           blk_eW1, blk_eb1, blk_eW2, blk_eb2,
           blk_nW1, blk_nb1, blk_nW2, blk_nb2,
           dec_W1, dec_b1, dec_W2, dec_b2):
    n = x.shape[0]
    ne = edge_attr.shape[0]
    nb = blk_eW1.shape[0]
    bn = 2000 if n % 2000 == 0 else n
    be = 4000 if ne % 4000 == 0 else ne
    src = edge_index[0]
    dst = edge_index[1]
    r = lambda b: b.reshape(1, -1)

    h = _encode_nodes(x, enc_n_W1, r(enc_n_b1), enc_n_W2, r(enc_n_b2), bn)
    e = _encode_edges(edge_attr, enc_e_W1, r(enc_e_b1),
                      enc_e_W2, r(enc_e_b2), be)

    use_sc = (ne % (_NC * _NS * _SCK) == 0) and (n % (_NS * 5) == 0)
    for i in range(nb):
        if use_sc:
            hs, hd = _sc_gather(h, src, dst)
        else:
            hs, hd = h[src], h[dst]
        eup, e = _edge_block(e, hs, hd, blk_eW1[i], r(blk_eb1[i]),
                             blk_eW2[i], r(blk_eb2[i]), be)
        agg = jax.ops.segment_sum(eup, dst, num_segments=n)
        h = _node_block(h, agg, blk_nW1[i], r(blk_nb1[i]),
                        blk_nW2[i], r(blk_nb2[i]), bn)

    w2p = jnp.zeros((_H, _H), jnp.float32).at[:, 0].set(dec_W2[:, 0])
    b2p = jnp.full((1, _H), dec_b2[0], jnp.float32)
    out = _decode(h, dec_W1, r(dec_b1), w2p, b2p, bn)
    return out[:, 0]
